# Initial kernel scaffold; baseline (speedup 1.0000x reference)
import jax
import jax.numpy as jnp
from jax.experimental import pallas as pl


def kernel(x):
    raise NotImplementedError("write your pallas kernel here")


# speedup vs baseline: 2.9404x; 2.9404x over previous
import functools

import jax
import jax.numpy as jnp
from jax import lax
from jax.experimental import pallas as pl
from jax.experimental.pallas import tpu as pltpu
from jax.experimental.pallas import tpu_sc as plsc

B, C, H, W = 4, 192, 224, 224
K, CELL = 3, 8
CS = C // K
OW = W * CELL
TH = 32
NCH = H // TH
NW = 32
UNITS = B * K * CELL
UPW = UNITS // NW
SLAB = TH * W


def _sc_body(x_hbm, out_hbm, in_v, out_v, sem):
    wid = lax.axis_index("s") * 2 + lax.axis_index("c")
    iota = lax.iota(jnp.int32, 16)

    def chunk_body(t, carry):
        unit = wid * UPW + t // NCH
        ci = t % NCH
        b = unit // (K * CELL)
        rem = unit % (K * CELL)
        k = rem // CELL
        p = rem % CELL
        bk = b * K + k
        c0 = k * CS + p * CELL
        h0 = ci * TH

        cps = [
            pltpu.async_copy(
                x_hbm.at[pl.ds(((b * C + c0 + q) * H + h0) * W, SLAB)],
                in_v.at[pl.ds(q * SLAB, SLAB)],
                sem,
            )
            for q in range(CELL)
        ]
        for cp in cps:
            cp.wait()

        def row_body(h, carry2):
            hb = h * OW
            for q in range(CELL):
                vb = iota * CELL + q
                src0 = q * SLAB + h * W
                for j in range(W // 16):
                    val = in_v[pl.ds(src0 + j * 16, 16)]
                    plsc.store_scatter(out_v, [vb + (hb + j * 128)], val)
            return carry2

        lax.fori_loop(0, TH, row_body, 0)

        obase = (bk * H + h0) * CELL + p
        ocs = [
            pltpu.async_copy(
                out_v.at[pl.ds(h * OW, OW)],
                out_hbm.at[pl.ds((obase + h * CELL) * OW, OW)],
                sem,
            )
            for h in range(TH)
        ]
        for cp in ocs:
            cp.wait()
        return carry

    lax.fori_loop(0, UPW * NCH, chunk_body, 0)


@functools.partial(
    pl.kernel,
    mesh=plsc.VectorSubcoreMesh(core_axis_name="c", subcore_axis_name="s"),
    out_type=jax.ShapeDtypeStruct((B * K * H * CELL * OW,), jnp.float32),
    scratch_types=[
        pltpu.VMEM((CELL * SLAB,), jnp.float32),
        pltpu.VMEM((TH * OW,), jnp.float32),
        pltpu.SemaphoreType.DMA,
    ],
    compiler_params=pltpu.CompilerParams(needs_layout_passes=False),
)
def _dts_sc(x_hbm, out_hbm, in_v, out_v, sem):
    _sc_body(x_hbm, out_hbm, in_v, out_v, sem)


def kernel(x):
    y = _dts_sc(x.reshape(-1))
    return y.reshape(B, K, H * CELL, OW)
